# Initial kernel scaffold; baseline (speedup 1.0000x reference)
import jax
import jax.numpy as jnp
from jax.experimental import pallas as pl


def kernel(x, edge_index, W, b, gamma, beta):
    raise NotImplementedError("write your pallas kernel here")


# speedup vs baseline: 1.4794x; 1.4794x over previous
import functools

import jax
import jax.numpy as jnp
from jax import lax
from jax.experimental import pallas as pl
from jax.experimental.pallas import tpu as pltpu
from jax.experimental.pallas import tpu_sc as plsc

_C = 128
_K = 32
_O = 128
_N = 10000
_L = 16

_NW = 32
_NPW = 320
_NPAD = _NW * _NPW
_GN = 2
_ROWS = 2 * _GN * _K
_G = _NPW // _GN
_NBUF = 4
_T = _G // _NBUF
_SLAB = (_NPW + 2 * _NBUF) * 2 * _K
_EPAD = (_NPAD + 16) * 2 * _K


def _sc_gather_max(table, eflat):
    mesh = plsc.VectorSubcoreMesh(core_axis_name="c", subcore_axis_name="s")

    @functools.partial(
        pl.kernel,
        out_type=jax.ShapeDtypeStruct((_NPAD, _C), jnp.float32),
        mesh=mesh,
        scratch_types=[
            pltpu.VMEM((_SLAB,), jnp.int32),
            *[pltpu.VMEM((_ROWS, _C), jnp.float32) for _ in range(_NBUF)],
            pltpu.VMEM((_GN, _C), jnp.float32),
            *[pltpu.SemaphoreType.DMA for _ in range(_NBUF)],
        ],
    )
    def k(table_hbm, idx_hbm, out_hbm,
          slab, buf0, buf1, buf2, buf3, outb, s0, s1, s2, s3):
        bufs = (buf0, buf1, buf2, buf3)
        sems = (s0, s1, s2, s3)
        wid = lax.axis_index("s") * 2 + lax.axis_index("c")
        nbase = wid * _NPW
        pltpu.sync_copy(idx_hbm.at[pl.ds(nbase * 2 * _K, _SLAB)], slab)

        def gather(g, bi):
            return pltpu.make_async_copy(
                table_hbm.at[slab.at[pl.ds(g * _ROWS, _ROWS)]],
                bufs[bi], sems[bi])

        for bi in range(_NBUF):
            gather(bi, bi).start()

        def tbody(t, carry):
            for bi in range(_NBUF):
                g = t * _NBUF + bi
                gather(g, bi).wait()
                buf = bufs[bi]

                def cbody(i, c2, buf=buf):
                    nn = i // (_C // _L)
                    cc = (i % (_C // _L)) * _L
                    rb = nn * 2 * _K
                    acc = buf[rb, pl.ds(cc, _L)] - buf[rb + _K, pl.ds(cc, _L)]
                    for kk in range(1, _K):
                        acc = jnp.maximum(
                            acc,
                            buf[rb + kk, pl.ds(cc, _L)]
                            - buf[rb + _K + kk, pl.ds(cc, _L)])
                    outb[nn, pl.ds(cc, _L)] = acc
                    return c2

                lax.fori_loop(0, _GN * (_C // _L), cbody, 0)
                pltpu.sync_copy(outb, out_hbm.at[pl.ds(nbase + g * _GN, _GN)])
                gather(g + _NBUF, bi).start()
            return carry

        lax.fori_loop(0, _T, tbody, 0)
        for bi in range(_NBUF):
            gather(_T * _NBUF + bi, bi).wait()

    return k(table, eflat)


def _tc_head(x3, m, W1, W2, b2, g2, be2):
    def body(x3_ref, m_ref, w1_ref, w2_ref, b_ref, g_ref, be_ref, out_ref):
        x3v = x3_ref[...]
        mv = m_ref[0:_N, :]
        w1 = w1_ref[...]
        w2 = w2_ref[...]
        y = lax.dot_general(w1, x3v, (((1,), (0,)), ((), ())),
                            preferred_element_type=jnp.float32)
        y = y + lax.dot_general(w2, mv, (((1,), (1,)), ((), ())),
                                preferred_element_type=jnp.float32)
        y = y + b_ref[...]
        mu = jnp.mean(y, axis=1, keepdims=True)
        yc = y - mu
        var = jnp.mean(yc * yc, axis=1, keepdims=True)
        yn = yc * lax.rsqrt(var + 1e-5)
        yn = yn * g_ref[...] + be_ref[...]
        out_ref[...] = 0.5 * yn * (1.0 + lax.erf(yn * 0.7071067811865476))

    return pl.pallas_call(
        body,
        out_shape=jax.ShapeDtypeStruct((_O, _N), jnp.float32),
    )(x3, m, W1, W2, b2, g2, be2)


def kernel(x, edge_index, W, b, gamma, beta):
    x3 = x[0, :, :, 0]
    table = x3.T
    e0 = edge_index[0, 0]
    e1 = edge_index[1, 0]
    ef = jnp.concatenate([e0, e1], axis=1).reshape(-1)
    ef = jnp.pad(ef, (0, _EPAD - ef.shape[0]))
    m = _sc_gather_max(table, ef)
    y = _tc_head(x3, m, W[:, 0::2], W[:, 1::2],
                 b.reshape(_O, 1), gamma.reshape(_O, 1), beta.reshape(_O, 1))
    return y.reshape(1, _O, _N, 1)


# speedup vs baseline: 1.4803x; 1.0006x over previous
import functools

import jax
import jax.numpy as jnp
from jax import lax
from jax.experimental import pallas as pl
from jax.experimental.pallas import tpu as pltpu
from jax.experimental.pallas import tpu_sc as plsc

_C = 128
_K = 32
_O = 128
_N = 10000
_L = 16

_NW = 32
_NPW = 320
_NPAD = _NW * _NPW
_GN = 2
_ROWS = 2 * _GN * _K
_G = _NPW // _GN
_NBUF = 4
_T = _G // _NBUF
_SLAB = (_NPW + 2 * _NBUF) * 2 * _K
_EPAD = (_NPAD + 16) * 2 * _K


def _sc_gather_max(table, eflat):
    mesh = plsc.VectorSubcoreMesh(core_axis_name="c", subcore_axis_name="s")

    @functools.partial(
        pl.kernel,
        out_type=jax.ShapeDtypeStruct((_NPAD, _C), jnp.float32),
        mesh=mesh,
        scratch_types=[
            pltpu.VMEM((_SLAB,), jnp.int32),
            *[pltpu.VMEM((_ROWS, _C), jnp.float32) for _ in range(_NBUF)],
            *[pltpu.VMEM((_GN, _C), jnp.float32) for _ in range(2)],
            *[pltpu.SemaphoreType.DMA for _ in range(_NBUF + 2)],
        ],
    )
    def k(table_hbm, idx_hbm, out_hbm,
          slab, buf0, buf1, buf2, buf3, outb0, outb1,
          s0, s1, s2, s3, os0, os1):
        bufs = (buf0, buf1, buf2, buf3)
        sems = (s0, s1, s2, s3)
        outbs = (outb0, outb1)
        osems = (os0, os1)
        wid = lax.axis_index("s") * 2 + lax.axis_index("c")
        nbase = wid * _NPW
        pltpu.sync_copy(idx_hbm.at[pl.ds(nbase * 2 * _K, _SLAB)], slab)

        def gather(g, bi):
            return pltpu.make_async_copy(
                table_hbm.at[slab.at[pl.ds(g * _ROWS, _ROWS)]],
                bufs[bi], sems[bi])

        for bi in range(_NBUF):
            gather(bi, bi).start()

        def flush(g, oi):
            return pltpu.make_async_copy(
                outbs[oi], out_hbm.at[pl.ds(nbase + g * _GN, _GN)], osems[oi])

        def tbody(t, carry):
            for bi in range(_NBUF):
                g = t * _NBUF + bi
                oi = bi % 2
                gather(g, bi).wait()
                buf = bufs[bi]
                outb = outbs[oi]

                @pl.when(g >= 2)
                def _():
                    flush(g - 2, oi).wait()

                def cbody(i, c2, buf=buf, outb=outb):
                    nn = i // (_C // _L)
                    cc = (i % (_C // _L)) * _L
                    rb = nn * 2 * _K
                    acc = buf[rb, pl.ds(cc, _L)] - buf[rb + _K, pl.ds(cc, _L)]
                    for kk in range(1, _K):
                        acc = jnp.maximum(
                            acc,
                            buf[rb + kk, pl.ds(cc, _L)]
                            - buf[rb + _K + kk, pl.ds(cc, _L)])
                    outb[nn, pl.ds(cc, _L)] = acc
                    return c2

                lax.fori_loop(0, _GN * (_C // _L), cbody, 0)
                flush(g, oi).start()
                gather(g + _NBUF, bi).start()
            return carry

        lax.fori_loop(0, _T, tbody, 0)
        for oi in range(2):
            flush(_G - 2 + oi, oi).wait()
        for bi in range(_NBUF):
            gather(_T * _NBUF + bi, bi).wait()

    return k(table, eflat)


def _tc_head(x3, m, W1, W2, b2, g2, be2):
    def body(x3_ref, m_ref, w1_ref, w2_ref, b_ref, g_ref, be_ref, out_ref):
        x3v = x3_ref[...]
        mv = m_ref[0:_N, :]
        w1 = w1_ref[...]
        w2 = w2_ref[...]
        y = lax.dot_general(w1, x3v, (((1,), (0,)), ((), ())),
                            preferred_element_type=jnp.float32)
        y = y + lax.dot_general(w2, mv, (((1,), (1,)), ((), ())),
                                preferred_element_type=jnp.float32)
        y = y + b_ref[...]
        mu = jnp.mean(y, axis=1, keepdims=True)
        yc = y - mu
        var = jnp.mean(yc * yc, axis=1, keepdims=True)
        yn = yc * lax.rsqrt(var + 1e-5)
        yn = yn * g_ref[...] + be_ref[...]
        out_ref[...] = 0.5 * yn * (1.0 + lax.erf(yn * 0.7071067811865476))

    return pl.pallas_call(
        body,
        out_shape=jax.ShapeDtypeStruct((_O, _N), jnp.float32),
    )(x3, m, W1, W2, b2, g2, be2)


def kernel(x, edge_index, W, b, gamma, beta):
    x3 = x[0, :, :, 0]
    table = x3.T
    e0 = edge_index[0, 0]
    e1 = edge_index[1, 0]
    ef = jnp.concatenate([e0, e1], axis=1).reshape(-1)
    ef = jnp.pad(ef, (0, _EPAD - ef.shape[0]))
    m = _sc_gather_max(table, ef)
    y = _tc_head(x3, m, W[:, 0::2], W[:, 1::2],
                 b.reshape(_O, 1), gamma.reshape(_O, 1), beta.reshape(_O, 1))
    return y.reshape(1, _O, _N, 1)


# speedup vs baseline: 6.6638x; 4.5018x over previous
import functools

import jax
import jax.numpy as jnp
from jax import lax
from jax.experimental import pallas as pl
from jax.experimental.pallas import tpu as pltpu
from jax.experimental.pallas import tpu_sc as plsc

_C = 128
_K = 32
_O = 128
_N = 10000
_L = 16

_NW = 32
_NPW = 320
_NPAD = _NW * _NPW
_GN = 1
_ROWS = 2 * _GN * _K
_G = _NPW // _GN
_NBUF = 4
_T = _G // _NBUF
_SLAB = (_NPW + 2 * _NBUF) * 2 * _K
_EPAD = (_NPAD + 16) * 2 * _K


_BL = 32
_TROWS = _NPAD // 16


def _sc_gather_max(table, eflat):
    mesh = plsc.VectorSubcoreMesh(core_axis_name="c", subcore_axis_name="s")

    @functools.partial(
        pl.kernel,
        out_type=jax.ShapeDtypeStruct((_NPAD * _C,), jnp.float32),
        mesh=mesh,
        scratch_types=[
            pltpu.VMEM_SHARED((_NPAD, _C), jnp.float32),
            *[pltpu.VMEM((_ROWS,), jnp.int32) for _ in range(_NBUF)],
            *[pltpu.VMEM((_ROWS, _C), jnp.float32) for _ in range(_NBUF)],
            *[pltpu.VMEM((2 * _C,), jnp.float32) for _ in range(2)],
            *[pltpu.SemaphoreType.DMA for _ in range(2 * _NBUF + 2)],
        ],
    )
    def k(table_hbm, idx_hbm, out_hbm,
          shtab, ib0, ib1, ib2, ib3, buf0, buf1, buf2, buf3, outb0, outb1,
          i0, i1, i2, i3, s0, s1, s2, s3, os0, os1):
        bufs = (buf0, buf1, buf2, buf3)
        idxbs = (ib0, ib1, ib2, ib3)
        isems = (i0, i1, i2, i3)
        sems = (s0, s1, s2, s3)
        outbs = (outb0, outb1)
        osems = (os0, os1)
        sid = lax.axis_index("s")
        wid = sid * 2 + lax.axis_index("c")
        nbase = wid * _NPW
        pltpu.sync_copy(table_hbm.at[pl.ds(sid * _TROWS, _TROWS)],
                        shtab.at[pl.ds(sid * _TROWS, _TROWS)])
        plsc.subcore_barrier()

        def ridx(g, bi):
            return pltpu.make_async_copy(
                idx_hbm.at[pl.ds((nbase + g) * _ROWS, _ROWS)],
                idxbs[bi], isems[bi])

        def gather(g, bi):
            return pltpu.make_async_copy(
                shtab.at[idxbs[bi]], bufs[bi], sems[bi])

        for bi in range(_NBUF):
            ridx(bi, bi).start()
        for bi in range(_NBUF):
            ridx(bi, bi).wait()
            gather(bi, bi).start()

        def flush(t, oi):
            return pltpu.make_async_copy(
                outbs[oi],
                out_hbm.at[pl.ds((nbase + t * _NBUF + oi * 2) * _C, 2 * _C)],
                osems[oi])

        def tbody(t, carry):
            for bi in range(_NBUF):
                g = t * _NBUF + bi
                oi = bi // 2
                half = bi % 2
                gather(g, bi).wait()
                ridx(g + _NBUF, bi).start()
                buf = bufs[bi]
                outb = outbs[oi]

                if half == 0:
                    @pl.when(t >= 1)
                    def _():
                        flush(t - 1, oi).wait()

                def cbody(i, c2, buf=buf, outb=outb, half=half):
                    cc = i * _L

                    def ld(r):
                        return buf[r, pl.ds(cc, _L)]

                    acc = ld(0) - ld(_K)
                    for kk in range(1, _K):
                        acc = jnp.maximum(acc, ld(kk) - ld(_K + kk))
                    outb[pl.ds(half * _C + cc, _L)] = acc
                    return c2

                lax.fori_loop(0, _C // _L, cbody, 0)
                if half == 1:
                    flush(t, oi).start()
                ridx(g + _NBUF, bi).wait()
                gather(g + _NBUF, bi).start()
            return carry

        lax.fori_loop(0, _T, tbody, 0)
        for oi in range(2):
            flush(_T - 1, oi).wait()
        for bi in range(_NBUF):
            gather(_T * _NBUF + bi, bi).wait()

    return k(table, eflat)


def _tc_head(x3, m, W1, W2, b2, g2, be2):
    def body(x3_ref, m_ref, w1_ref, w2_ref, b_ref, g_ref, be_ref, out_ref):
        x3v = x3_ref[...]
        mv = m_ref[0:_N, :]
        w1 = w1_ref[...]
        w2 = w2_ref[...]
        y = lax.dot_general(w1, x3v, (((1,), (0,)), ((), ())),
                            preferred_element_type=jnp.float32)
        y = y + lax.dot_general(w2, mv, (((1,), (1,)), ((), ())),
                                preferred_element_type=jnp.float32)
        y = y + b_ref[...]
        mu = jnp.mean(y, axis=1, keepdims=True)
        yc = y - mu
        var = jnp.mean(yc * yc, axis=1, keepdims=True)
        yn = yc * lax.rsqrt(var + 1e-5)
        yn = yn * g_ref[...] + be_ref[...]
        out_ref[...] = 0.5 * yn * (1.0 + lax.erf(yn * 0.7071067811865476))

    return pl.pallas_call(
        body,
        out_shape=jax.ShapeDtypeStruct((_O, _N), jnp.float32),
    )(x3, m, W1, W2, b2, g2, be2)


def kernel(x, edge_index, W, b, gamma, beta):
    x3 = x[0, :, :, 0]
    table = jnp.pad(x3.T, ((0, _NPAD - _N), (0, 0)))
    e0 = edge_index[0, 0]
    e1 = edge_index[1, 0]
    ef = jnp.concatenate([e0, e1], axis=1).reshape(-1)
    ef = jnp.pad(ef, (0, _EPAD - ef.shape[0]))
    m = _sc_gather_max(table, ef).reshape(_NPAD, _C)
    y = _tc_head(x3, m, W[:, 0::2], W[:, 1::2],
                 b.reshape(_O, 1), gamma.reshape(_O, 1), beta.reshape(_O, 1))
    return y.reshape(1, _O, _N, 1)


# speedup vs baseline: 6.8086x; 1.0217x over previous
import functools

import jax
import jax.numpy as jnp
from jax import lax
from jax.experimental import pallas as pl
from jax.experimental.pallas import tpu as pltpu
from jax.experimental.pallas import tpu_sc as plsc

_C = 128
_K = 32
_O = 128
_N = 10000
_L = 16

_NW = 32
_NPW = 320
_NPAD = _NW * _NPW
_GN = 1
_ROWS = 2 * _GN * _K
_G = _NPW // _GN
_NBUF = 5
_T = _G // _NBUF
_SLAB = (_NPW + 2 * _NBUF) * 2 * _K
_EPAD = (_NPAD + 16) * 2 * _K


_TSH = 632


def _sc_gather_max(table, eflat):
    mesh = plsc.VectorSubcoreMesh(core_axis_name="c", subcore_axis_name="s")

    @functools.partial(
        pl.kernel,
        out_type=jax.ShapeDtypeStruct((_NPAD * _C,), jnp.float32),
        mesh=mesh,
        scratch_types=[
            pltpu.VMEM_SHARED((_N, _C), jnp.float32),
            *[pltpu.VMEM((_ROWS,), jnp.int32) for _ in range(_NBUF)],
            *[pltpu.VMEM((_ROWS, _C), jnp.float32) for _ in range(_NBUF)],
            *[pltpu.VMEM((_C,), jnp.float32) for _ in range(_NBUF)],
            *[pltpu.SemaphoreType.DMA for _ in range(3 * _NBUF)],
        ],
    )
    def k(table_hbm, idx_hbm, out_hbm,
          shtab, ib0, ib1, ib2, ib3, ib4, buf0, buf1, buf2, buf3, buf4,
          ob0, ob1, ob2, ob3, ob4,
          i0, i1, i2, i3, i4, s0, s1, s2, s3, s4, o0, o1, o2, o3, o4):
        bufs = (buf0, buf1, buf2, buf3, buf4)
        idxbs = (ib0, ib1, ib2, ib3, ib4)
        isems = (i0, i1, i2, i3, i4)
        sems = (s0, s1, s2, s3, s4)
        outbs = (ob0, ob1, ob2, ob3, ob4)
        osems = (o0, o1, o2, o3, o4)
        sid = lax.axis_index("s")
        wid = sid * 2 + lax.axis_index("c")
        nbase = wid * _NPW
        @pl.when(sid < 15)
        def _():
            pltpu.sync_copy(table_hbm.at[pl.ds(sid * _TSH, _TSH)],
                            shtab.at[pl.ds(sid * _TSH, _TSH)])

        @pl.when(sid == 15)
        def _():
            pltpu.sync_copy(table_hbm.at[pl.ds(15 * _TSH, _N - 15 * _TSH)],
                            shtab.at[pl.ds(15 * _TSH, _N - 15 * _TSH)])

        plsc.subcore_barrier()

        def ridx(g, bi):
            return pltpu.make_async_copy(
                idx_hbm.at[pl.ds((nbase + g) * _ROWS, _ROWS)],
                idxbs[bi], isems[bi])

        def gather(g, bi):
            return pltpu.make_async_copy(
                shtab.at[idxbs[bi]], bufs[bi], sems[bi])

        for bi in range(_NBUF):
            ridx(bi, bi).start()
        for bi in range(_NBUF):
            ridx(bi, bi).wait()
            gather(bi, bi).start()

        def flush(g, bi):
            return pltpu.make_async_copy(
                outbs[bi],
                out_hbm.at[pl.ds((nbase + g) * _C, _C)],
                osems[bi])

        def tbody(t, carry):
            for bi in range(_NBUF):
                g = t * _NBUF + bi
                gather(g, bi).wait()
                ridx(g + _NBUF, bi).start()
                buf = bufs[bi]
                outb = outbs[bi]

                @pl.when(t >= 1)
                def _():
                    flush(g - _NBUF, bi).wait()

                def cbody(i, c2, buf=buf, outb=outb):
                    cc = i * _L

                    def ld(r):
                        return buf[r, pl.ds(cc, _L)]

                    acc = ld(0) - ld(_K)
                    for kk in range(1, _K):
                        acc = jnp.maximum(acc, ld(kk) - ld(_K + kk))
                    outb[pl.ds(cc, _L)] = acc
                    return c2

                lax.fori_loop(0, _C // _L, cbody, 0)
                flush(g, bi).start()
                ridx(g + _NBUF, bi).wait()
                gather(g + _NBUF, bi).start()
            return carry

        lax.fori_loop(0, _T, tbody, 0)
        for bi in range(_NBUF):
            flush((_T - 1) * _NBUF + bi, bi).wait()
        for bi in range(_NBUF):
            gather(_T * _NBUF + bi, bi).wait()

    return k(table, eflat)


def _tc_head(x3, m, W1, W2, b2, g2, be2):
    def body(x3_ref, m_ref, w1_ref, w2_ref, b_ref, g_ref, be_ref, out_ref):
        x3v = x3_ref[...]
        mv = m_ref[0:_N, :]
        w1 = w1_ref[...]
        w2 = w2_ref[...]
        y = lax.dot_general(w1, x3v, (((1,), (0,)), ((), ())),
                            preferred_element_type=jnp.float32)
        y = y + lax.dot_general(w2, mv, (((1,), (1,)), ((), ())),
                                preferred_element_type=jnp.float32)
        y = y + b_ref[...]
        mu = jnp.mean(y, axis=1, keepdims=True)
        yc = y - mu
        var = jnp.mean(yc * yc, axis=1, keepdims=True)
        yn = yc * lax.rsqrt(var + 1e-5)
        yn = yn * g_ref[...] + be_ref[...]
        out_ref[...] = 0.5 * yn * (1.0 + lax.erf(yn * 0.7071067811865476))

    return pl.pallas_call(
        body,
        out_shape=jax.ShapeDtypeStruct((_O, _N), jnp.float32),
    )(x3, m, W1, W2, b2, g2, be2)


def kernel(x, edge_index, W, b, gamma, beta):
    x3 = x[0, :, :, 0]
    table = x3.T
    e0 = edge_index[0, 0]
    e1 = edge_index[1, 0]
    ef = jnp.concatenate([e0, e1], axis=1).reshape(-1)
    ef = jnp.pad(ef, (0, _EPAD - ef.shape[0]))
    m = _sc_gather_max(table, ef).reshape(_NPAD, _C)
    y = _tc_head(x3, m, W[:, 0::2], W[:, 1::2],
                 b.reshape(_O, 1), gamma.reshape(_O, 1), beta.reshape(_O, 1))
    return y.reshape(1, _O, _N, 1)
